# baseline (speedup 1.0000x reference)
import functools

import jax
import jax.numpy as jnp
from jax import lax
from jax.experimental import pallas as pl
from jax.experimental.pallas import tpu as pltpu
from jax.experimental.pallas import tpu_sc as plsc

N_E = 8192
E_DIM = 32
B = 8192
BLK = 1024
NCHUNK = N_E // BLK
NW = 32
PERW = B // NW


def _k1_body(z_ref, et2_ref, idx_ref, loss_ref, sse_ref):
    i = pl.program_id(0)
    z = z_ref[...]
    zsq = z * z
    for h in (16, 8, 4, 2, 1):
        zsq = zsq[:, :h] + zsq[:, h:2 * h]
    zb = z.astype(jnp.bfloat16).astype(jnp.float32)
    vals, idxs = [], []
    for j in range(NCHUNK):
        et2 = et2_ref[:, pl.ds(j * BLK, BLK)]
        m2 = lax.dot_general(zb, et2, (((1,), (0,)), ((), ())),
                             preferred_element_type=jnp.float32)
        esq = 0.25 * jnp.sum(et2 * et2, axis=0, keepdims=True)
        d = (zsq + esq) - m2
        lm = jnp.min(d, axis=1, keepdims=True)
        la = (jnp.argmin(d, axis=1, keepdims=True).astype(jnp.int32)
              + (j * BLK))
        vals.append(lm)
        idxs.append(la)

    def first_min(v0, i0, v1, i1):
        upd = v1 < v0
        return jnp.where(upd, v1, v0), jnp.where(upd, i1, i0)

    def comb(va, ia, vb, ib):
        aw = va.astype(jnp.bfloat16).astype(jnp.float32)
        tb = vb < aw
        return jnp.where(tb, vb, va), jnp.where(tb, ib, ia)

    qv, qi = [], []
    for q in range(4):
        v, i2 = first_min(vals[2 * q], idxs[2 * q], vals[2 * q + 1], idxs[2 * q + 1])
        qv.append(v)
        qi.append(i2)
    v0, i0 = comb(qv[0], qi[0], qv[1], qi[1])
    v1, i1 = comb(qv[2], qi[2], qv[3], qi[3])
    acc_v, acc_i = comb(v0, i0, v1, i1)
    idx_ref[...] = acc_i

    @pl.when(i == 0)
    def _():
        sse_ref[0, 0] = 0.0

    sse_ref[0, 0] += jnp.sum(acc_v)

    @pl.when(i == pl.num_programs(0) - 1)
    def _():
        loss_ref[...] = jnp.broadcast_to(
            1.25 * sse_ref[0, 0] / float(B * E_DIM), (1, 1))


_k1 = pl.pallas_call(
    _k1_body,
    grid=(B // BLK,),
    in_specs=[
        pl.BlockSpec((BLK, E_DIM), lambda i: (i, 0)),
        pl.BlockSpec((E_DIM, N_E), lambda i: (0, 0)),
    ],
    out_specs=[
        pl.BlockSpec((BLK, 1), lambda i: (i, 0)),
        pl.BlockSpec((1, 1), lambda i: (0, 0)),
    ],
    out_shape=[
        jax.ShapeDtypeStruct((B, 1), jnp.int32),
        jax.ShapeDtypeStruct((1, 1), jnp.float32),
    ],
    scratch_shapes=[pltpu.SMEM((1, 1), jnp.float32)],
)


def _k2_body(emb_hbm, idx_hbm, zeros_hbm, zq_hbm, cnt_hbm,
             idx_v, rows_v, ones_v, shared, sem):
    c = lax.axis_index("c")
    s = lax.axis_index("s")
    wid = s * 2 + c
    base = wid * PERW
    pltpu.sync_copy(idx_hbm.at[pl.ds(wid * 2, 2)], idx_v)
    for t in range(2):
        pltpu.async_copy(emb_hbm.at[idx_v.at[t]],
                         rows_v.at[pl.ds(t * 128, 128)], sem).wait()
    pltpu.sync_copy(rows_v, zq_hbm.at[pl.ds(base, PERW)])

    for t in range(2):
        for u in range(8):
            ones_v[t, pl.ds(u * 16, 16)] = jnp.ones((16,), jnp.float32)

    @pl.when(s == 0)
    def _():
        pltpu.sync_copy(zeros_hbm, shared)

    plsc.subcore_barrier()
    for t in range(2):
        pltpu.sync_copy(ones_v.at[t], shared.at[idx_v.at[t]], add=True)
    plsc.subcore_barrier()

    @pl.when(s == 0)
    def _():
        pltpu.sync_copy(shared, cnt_hbm.at[c])


@functools.cache
def _k2():
    return functools.partial(
        pl.kernel,
        out_type=[
            jax.ShapeDtypeStruct((B, E_DIM), jnp.float32),
            jax.ShapeDtypeStruct((2, N_E), jnp.float32),
        ],
        mesh=plsc.VectorSubcoreMesh(core_axis_name="c", subcore_axis_name="s"),
        compiler_params=pltpu.CompilerParams(use_tc_tiling_on_sc=False),
        scratch_types=[
            pltpu.VMEM((2, 128), jnp.int32),
            pltpu.VMEM((PERW, E_DIM), jnp.float32),
            pltpu.VMEM((2, 128), jnp.float32),
            pltpu.VMEM_SHARED((N_E,), jnp.float32),
            pltpu.SemaphoreType.DMA,
        ],
    )(_k2_body)


def _k3_body(cnt_ref, perp_ref):
    counts = cnt_ref[0:1, :] + cnt_ref[1:2, :]
    avg = counts * (1.0 / float(B))
    ent = jnp.sum(avg * jnp.log(avg + 1e-12))
    perp_ref[...] = jnp.broadcast_to(jnp.exp(-ent), (1, 1))


_k3 = pl.pallas_call(
    _k3_body,
    out_shape=jax.ShapeDtypeStruct((1, 1), jnp.float32),
)


def kernel(z, embedding):
    z_flat = z.reshape(B, E_DIM)
    e_t2 = 2.0 * embedding.T
    idx_col, loss = _k1(z_flat, e_t2)
    idx = idx_col.reshape(B)
    zq_flat, counts2 = _k2()(embedding, idx.reshape(NW * 2, 128),
                             jnp.zeros((N_E,), jnp.float32))
    perp = _k3(counts2)
    return (zq_flat.reshape(z.shape), loss.reshape(()), idx,
            perp.reshape(()))


# speedup vs baseline: 1.1817x; 1.1817x over previous
import functools

import jax
import jax.numpy as jnp
from jax import lax
from jax.experimental import pallas as pl
from jax.experimental.pallas import tpu as pltpu
from jax.experimental.pallas import tpu_sc as plsc

N_E = 8192
E_DIM = 32
B = 8192
BLK = 1024
NCHUNK = N_E // BLK
NW = 32
PERW = B // NW


def _k1_body(z_ref, et2_ref, idx_ref, loss_ref, sse_ref):
    i = pl.program_id(0)
    z = z_ref[...]
    zsq = z * z
    for h in (16, 8, 4, 2, 1):
        zsq = zsq[:, :h] + zsq[:, h:2 * h]
    zb = z.astype(jnp.bfloat16).astype(jnp.float32)
    ids0 = lax.broadcasted_iota(jnp.int32, (BLK, BLK), 1)
    vals, idxs = [], []
    for j in range(NCHUNK):
        et2 = et2_ref[:, pl.ds(j * BLK, BLK)]
        m2 = lax.dot_general(zb, et2, (((1,), (0,)), ((), ())),
                             preferred_element_type=jnp.float32)
        esq = 0.25 * jnp.sum(et2 * et2, axis=0, keepdims=True)
        d = (zsq + esq) - m2
        lm = jnp.min(d, axis=1, keepdims=True)
        cand = jnp.where(d == lm, ids0, jnp.int32(2**30))
        la = jnp.min(cand, axis=1, keepdims=True) + (j * BLK)
        vals.append(lm)
        idxs.append(la)

    def first_min(v0, i0, v1, i1):
        upd = v1 < v0
        return jnp.where(upd, v1, v0), jnp.where(upd, i1, i0)

    def comb(va, ia, vb, ib):
        aw = va.astype(jnp.bfloat16).astype(jnp.float32)
        tb = vb < aw
        return jnp.where(tb, vb, va), jnp.where(tb, ib, ia)

    qv, qi = [], []
    for q in range(4):
        v, i2 = first_min(vals[2 * q], idxs[2 * q], vals[2 * q + 1], idxs[2 * q + 1])
        qv.append(v)
        qi.append(i2)
    v0, i0 = comb(qv[0], qi[0], qv[1], qi[1])
    v1, i1 = comb(qv[2], qi[2], qv[3], qi[3])
    acc_v, acc_i = comb(v0, i0, v1, i1)
    idx_ref[...] = acc_i

    @pl.when(i == 0)
    def _():
        sse_ref[0, 0] = 0.0

    sse_ref[0, 0] += jnp.sum(acc_v)

    @pl.when(i == pl.num_programs(0) - 1)
    def _():
        loss_ref[...] = jnp.broadcast_to(
            1.25 * sse_ref[0, 0] / float(B * E_DIM), (1, 1))


_k1 = pl.pallas_call(
    _k1_body,
    grid=(B // BLK,),
    in_specs=[
        pl.BlockSpec((BLK, E_DIM), lambda i: (i, 0)),
        pl.BlockSpec((E_DIM, N_E), lambda i: (0, 0)),
    ],
    out_specs=[
        pl.BlockSpec((BLK, 1), lambda i: (i, 0)),
        pl.BlockSpec((1, 1), lambda i: (0, 0)),
    ],
    out_shape=[
        jax.ShapeDtypeStruct((B, 1), jnp.int32),
        jax.ShapeDtypeStruct((1, 1), jnp.float32),
    ],
    scratch_shapes=[pltpu.SMEM((1, 1), jnp.float32)],
)


def _k2_body(emb_hbm, idx_hbm, zeros_hbm, zq_hbm, cnt_hbm,
             idx_v, rows_v, ones_v, shared, sem):
    c = lax.axis_index("c")
    s = lax.axis_index("s")
    wid = s * 2 + c
    base = wid * PERW
    pltpu.sync_copy(idx_hbm.at[pl.ds(wid * 2, 2)], idx_v)
    for t in range(2):
        pltpu.async_copy(emb_hbm.at[idx_v.at[t]],
                         rows_v.at[pl.ds(t * 128, 128)], sem).wait()
    pltpu.sync_copy(rows_v, zq_hbm.at[pl.ds(base, PERW)])

    for t in range(2):
        for u in range(8):
            ones_v[t, pl.ds(u * 16, 16)] = jnp.ones((16,), jnp.float32)

    @pl.when(s == 0)
    def _():
        pltpu.sync_copy(zeros_hbm, shared)

    plsc.subcore_barrier()
    for t in range(2):
        pltpu.sync_copy(ones_v.at[t], shared.at[idx_v.at[t]], add=True)
    plsc.subcore_barrier()

    @pl.when(s == 0)
    def _():
        pltpu.sync_copy(shared, cnt_hbm.at[c])


@functools.cache
def _k2():
    return functools.partial(
        pl.kernel,
        out_type=[
            jax.ShapeDtypeStruct((B, E_DIM), jnp.float32),
            jax.ShapeDtypeStruct((2, N_E), jnp.float32),
        ],
        mesh=plsc.VectorSubcoreMesh(core_axis_name="c", subcore_axis_name="s"),
        compiler_params=pltpu.CompilerParams(use_tc_tiling_on_sc=False),
        scratch_types=[
            pltpu.VMEM((2, 128), jnp.int32),
            pltpu.VMEM((PERW, E_DIM), jnp.float32),
            pltpu.VMEM((2, 128), jnp.float32),
            pltpu.VMEM_SHARED((N_E,), jnp.float32),
            pltpu.SemaphoreType.DMA,
        ],
    )(_k2_body)


def _k3_body(cnt_ref, perp_ref):
    counts = cnt_ref[0:1, :] + cnt_ref[1:2, :]
    avg = counts * (1.0 / float(B))
    ent = jnp.sum(avg * jnp.log(avg + 1e-12))
    perp_ref[...] = jnp.broadcast_to(jnp.exp(-ent), (1, 1))


_k3 = pl.pallas_call(
    _k3_body,
    out_shape=jax.ShapeDtypeStruct((1, 1), jnp.float32),
)


def kernel(z, embedding):
    z_flat = z.reshape(B, E_DIM)
    e_t2 = 2.0 * embedding.T
    idx_col, loss = _k1(z_flat, e_t2)
    idx = idx_col.reshape(B)
    zq_flat, counts2 = _k2()(embedding, idx.reshape(NW * 2, 128),
                             jnp.zeros((N_E,), jnp.float32))
    perp = _k3(counts2)
    return (zq_flat.reshape(z.shape), loss.reshape(()), idx,
            perp.reshape(()))
